# baseline (speedup 1.0000x reference)
import jax
import jax.numpy as jnp
from jax.experimental import pallas as pl
from jax.experimental.pallas import tpu as pltpu

_NEG_RATIO = 3.0


def _loss_body(pred_ref, gt_ref, mask_ref, neg_ref, stat_ref):
    p = pred_ref[0, 0, :, :]
    g = gt_ref[0, :, :]
    m = mask_ref[0, :, :]
    loss = jnp.abs(p - g)
    neg_ref[0, :, :] = loss * (1.0 - m)
    psum = jnp.sum(loss * m)
    pcnt = jnp.sum(m)
    lane = jax.lax.broadcasted_iota(jnp.int32, (1, 1, 128), 2)
    stat_ref[...] = jnp.where(lane == 0, psum,
                              jnp.where(lane == 1, pcnt, 0.0))


def kernel(pred, gt, mask):
    B = pred.shape[0]
    neg, stats = pl.pallas_call(
        _loss_body,
        grid=(B,),
        in_specs=[
            pl.BlockSpec((1, 1, 512, 512), lambda i: (i, 0, 0, 0)),
            pl.BlockSpec((1, 512, 512), lambda i: (i, 0, 0)),
            pl.BlockSpec((1, 512, 512), lambda i: (i, 0, 0)),
        ],
        out_specs=[
            pl.BlockSpec((1, 512, 512), lambda i: (i, 0, 0)),
            pl.BlockSpec((1, 1, 128), lambda i: (i, 0, 0)),
        ],
        out_shape=[
            jax.ShapeDtypeStruct((B, 512, 512), jnp.float32),
            jax.ShapeDtypeStruct((B, 1, 128), jnp.float32),
        ],
    )(pred, gt, mask)

    pos_sum = stats[:, 0, 0].sum()
    pos_cnt = jnp.floor(stats[:, 0, 1].sum())
    n_total = B * 512 * 512
    neg_cnt = jnp.minimum(jnp.floor(float(n_total) - stats[:, 0, 1].sum()),
                          jnp.floor(pos_cnt * _NEG_RATIO))

    negative_flat = neg.reshape(-1)
    negative_sorted, _ = jax.lax.top_k(negative_flat, n_total)
    idx = jnp.arange(n_total, dtype=jnp.float32)
    negative_top = jnp.where(idx < neg_cnt, negative_sorted, 0.0)
    negative_loss = negative_top.sum() / neg_cnt
    positive_loss = pos_sum / pos_cnt
    total = positive_loss + negative_loss
    return (total, positive_loss, negative_loss)


# speedup vs baseline: 17.9613x; 17.9613x over previous
import functools

import jax
import jax.numpy as jnp
from jax import lax
from jax.experimental import pallas as pl
from jax.experimental.pallas import tpu as pltpu
from jax.experimental.pallas import tpu_sc as plsc

_NEG_RATIO = 3.0

_NC = 2
_NS = 16
_LANE = 16
_NW = _NC * _NS

_NB = 2048
_TBL = _NB * _LANE

_N = 16 * 512 * 512
_PW = _N // _NW
_CH = 8192
_NCH = _PW // _CH


def _loss_body(pred_ref, gt_ref, mask_ref, neg_ref, stat_ref):
    p = pred_ref[0, 0, :, :]
    g = gt_ref[0, :, :]
    m = mask_ref[0, :, :]
    loss = jnp.abs(p - g)
    neg_ref[0, :, :] = loss * (1.0 - m)
    psum = jnp.sum(loss * m)
    pcnt = jnp.sum(m)
    lane = lax.broadcasted_iota(jnp.int32, (1, 1, 128), 2)
    stat_ref[...] = jnp.where(lane == 0, psum,
                              jnp.where(lane == 1, pcnt, 0.0))


def _hist_body(neg_hbm, par_hbm, cnt_hbm, sum_hbm,
               buf0, buf1, par_v, cnt_v, sum_v, sem0, sem1):
    wid = lax.axis_index("s") * _NC + lax.axis_index("c")

    pltpu.sync_copy(par_hbm, par_v)
    lo = par_v[pl.ds(0, _LANE)]
    width = par_v[pl.ds(_LANE, _LANE)]
    shift = par_v[pl.ds(2 * _LANE, _LANE)]

    zero = jnp.zeros((_LANE,), jnp.float32)

    def _zero(i, carry):
        cnt_v[pl.ds(i * _LANE, _LANE)] = zero
        sum_v[pl.ds(i * _LANE, _LANE)] = zero
        return carry

    lax.fori_loop(0, _TBL // _LANE, _zero, 0)

    lane = lax.iota(jnp.int32, _LANE)
    ones = jnp.ones((_LANE,), jnp.float32)
    sixteen = jnp.full((_LANE,), _LANE, jnp.int32)
    izero = jnp.zeros((_LANE,), jnp.int32)

    base = wid * _PW
    bufs = (buf0, buf1)
    sems = (sem0, sem1)

    def _chunk(buf):
        def _vreg(j, carry):
            v = buf[pl.ds(j * _LANE, _LANE)]
            bits = lax.bitcast_convert_type(v, jnp.int32)
            rel = bits - lo
            inr = (rel >= izero) & (rel < width)
            fb = lax.shift_right_logical(rel, shift)
            fb = jnp.where(inr, fb, izero)
            idx = fb * sixteen + lane
            plsc.addupdate_scatter(cnt_v, [idx], ones, mask=inr)
            plsc.addupdate_scatter(sum_v, [idx], v, mask=inr)
            return carry

        lax.fori_loop(0, _CH // _LANE, _vreg, 0)

    cur = pltpu.async_copy(neg_hbm.at[pl.ds(base, _CH)], buf0, sem0)
    for c in range(_NCH):
        nxt = None
        if c + 1 < _NCH:
            nxt = pltpu.async_copy(
                neg_hbm.at[pl.ds(base + (c + 1) * _CH, _CH)],
                bufs[(c + 1) % 2], sems[(c + 1) % 2])
        cur.wait()
        _chunk(bufs[c % 2])
        cur = nxt

    pltpu.sync_copy(cnt_v, cnt_hbm.at[wid])
    pltpu.sync_copy(sum_v, sum_hbm.at[wid])


_sc_hist = pl.kernel(
    _hist_body,
    out_type=[
        jax.ShapeDtypeStruct((_NW, _TBL), jnp.float32),
        jax.ShapeDtypeStruct((_NW, _TBL), jnp.float32),
    ],
    mesh=plsc.VectorSubcoreMesh(core_axis_name="c", subcore_axis_name="s"),
    compiler_params=pltpu.CompilerParams(needs_layout_passes=False),
    scratch_types=[
        pltpu.VMEM((_CH,), jnp.float32),
        pltpu.VMEM((_CH,), jnp.float32),
        pltpu.VMEM((3 * _LANE,), jnp.int32),
        pltpu.VMEM((_TBL,), jnp.float32),
        pltpu.VMEM((_TBL,), jnp.float32),
        pltpu.SemaphoreType.DMA,
        pltpu.SemaphoreType.DMA,
    ],
)


def _params(lo, width, shift):
    return jnp.concatenate([
        jnp.full((_LANE,), lo, jnp.int32),
        jnp.full((_LANE,), width, jnp.int32),
        jnp.full((_LANE,), shift, jnp.int32),
    ])


def _merge(tbl):
    return tbl.reshape(_NW, _NB, _LANE).sum(axis=(0, 2))


def _rev_cumsum(x):
    return jnp.cumsum(x[::-1])[::-1]


def kernel(pred, gt, mask):
    B = pred.shape[0]
    neg, stats = pl.pallas_call(
        _loss_body,
        grid=(B,),
        in_specs=[
            pl.BlockSpec((1, 1, 512, 512), lambda i: (i, 0, 0, 0)),
            pl.BlockSpec((1, 512, 512), lambda i: (i, 0, 0)),
            pl.BlockSpec((1, 512, 512), lambda i: (i, 0, 0)),
        ],
        out_specs=[
            pl.BlockSpec((1, 512, 512), lambda i: (i, 0, 0)),
            pl.BlockSpec((1, 1, 128), lambda i: (i, 0, 0)),
        ],
        out_shape=[
            jax.ShapeDtypeStruct((B, 512, 512), jnp.float32),
            jax.ShapeDtypeStruct((B, 1, 128), jnp.float32),
        ],
    )(pred, gt, mask)

    pos_sum = stats[:, 0, 0].sum()
    pos_cnt = jnp.floor(stats[:, 0, 1].sum())
    neg_cnt = jnp.minimum(jnp.floor(float(_N) - stats[:, 0, 1].sum()),
                          jnp.floor(pos_cnt * _NEG_RATIO))

    neg_flat = neg.reshape(_N)
    bins = jnp.arange(_NB, dtype=jnp.int32)

    cnt_o, sum_o = _sc_hist(neg_flat, _params(0, 0x7FFFFFFF, 20))
    cnt1, sum1 = _merge(cnt_o), _merge(sum_o)
    h1 = _rev_cumsum(cnt1)
    b_star = jnp.max(jnp.where(h1 >= neg_cnt, bins, 0))
    ca = h1[b_star] - cnt1[b_star]
    s_above = _rev_cumsum(sum1)[b_star] - sum1[b_star]

    lo = b_star << 20
    fcnt_o, fsum_o = _sc_hist(neg_flat, _params(lo, 1 << 20, 9))
    fcnt, fsum = _merge(fcnt_o), _merge(fsum_o)
    hf = _rev_cumsum(fcnt)
    f_star = jnp.max(jnp.where(ca + hf >= neg_cnt, bins, 0))
    c_abv = ca + hf[f_star] - fcnt[f_star]
    s_abv = s_above + _rev_cumsum(fsum)[f_star] - fsum[f_star]
    deficit = neg_cnt - c_abv
    avg = fsum[f_star] / jnp.maximum(fcnt[f_star], 1.0)

    negative_loss = (s_abv + deficit * avg) / neg_cnt
    positive_loss = pos_sum / pos_cnt
    total = positive_loss + negative_loss
    return (total, positive_loss, negative_loss)
